# Initial kernel scaffold; baseline (speedup 1.0000x reference)
import functools

import jax
import jax.numpy as jnp
from jax import lax
from jax.experimental import pallas as pl
from jax.experimental.pallas import tpu as pltpu
from jax.experimental.pallas import tpu_sc as plsc

N = 10000
E = 320000
D = 128

NC = 2
NS = 16
NW = NC * NS

CH = 128
EPW = -(-E // (NW * CH)) * CH
E_PAD = EPW * NW
NCHUNK = EPW // CH

NPAD = N + 16
RPT = NPAD // NS

BLK = 400
GRID = N // BLK

_mesh = plsc.VectorSubcoreMesh(core_axis_name="c", subcore_axis_name="s")


@functools.partial(
    pl.kernel,
    out_type=jax.ShapeDtypeStruct((NC, NPAD, 16), jnp.float32),
    mesh=_mesh,
    scratch_types=[
        pltpu.VMEM((1, CH), jnp.int32),
        pltpu.VMEM((CH, 16), jnp.float32),
        pltpu.VMEM_SHARED((NPAD, 16), jnp.float32),
        pltpu.SemaphoreType.DMA,
    ],
)
def _deg_kernel(dst_hbm, ones_hbm, zeros_hbm, out_hbm, didx, ones_v, acc, sem):
    c = lax.axis_index("c")
    s = lax.axis_index("s")
    wid = c * NS + s
    pltpu.sync_copy(ones_hbm, ones_v)
    pltpu.sync_copy(zeros_hbm, acc.at[pl.ds(s * RPT, RPT), :])
    plsc.subcore_barrier()

    def body(ch, carry):
        off = wid * EPW + ch * CH
        pltpu.sync_copy(dst_hbm.at[pl.ds(off, CH)], didx.at[0])
        pltpu.sync_copy(ones_v, acc.at[didx.at[0]], add=True)
        return carry

    lax.fori_loop(0, NCHUNK, body, 0)
    plsc.subcore_barrier()
    pltpu.sync_copy(acc.at[pl.ds(s * RPT, RPT), :],
                    out_hbm.at[c, pl.ds(s * RPT, RPT), :])


@functools.partial(
    pl.kernel,
    out_type=jax.ShapeDtypeStruct((NC, NPAD, D), jnp.float32),
    mesh=_mesh,
    scratch_types=[
        pltpu.VMEM((1, CH), jnp.int32),
        pltpu.VMEM((1, CH), jnp.int32),
        pltpu.VMEM((CH, D), jnp.float32),
        pltpu.VMEM_SHARED((NPAD, D), jnp.float32),
        pltpu.SemaphoreType.DMA,
    ],
)
def _hop_kernel(g_hbm, src_hbm, dst_hbm, zeros_hbm, out_hbm,
                sidx, didx, rows, acc, sem):
    c = lax.axis_index("c")
    s = lax.axis_index("s")
    wid = c * NS + s
    pltpu.sync_copy(zeros_hbm, acc.at[pl.ds(s * RPT, RPT), :])
    plsc.subcore_barrier()

    def body(ch, carry):
        off = wid * EPW + ch * CH
        pltpu.sync_copy(src_hbm.at[pl.ds(off, CH)], sidx.at[0])
        pltpu.sync_copy(dst_hbm.at[pl.ds(off, CH)], didx.at[0])
        pltpu.async_copy(g_hbm.at[sidx.at[0]], rows, sem).wait()
        pltpu.sync_copy(rows, acc.at[didx.at[0]], add=True)
        return carry

    lax.fori_loop(0, NCHUNK, body, 0)
    plsc.subcore_barrier()
    pltpu.sync_copy(acc.at[pl.ds(s * RPT, RPT), :],
                    out_hbm.at[c, pl.ds(s * RPT, RPT), :])


def _scale_init_body(x_ref, degp_ref, g0_ref, dinv_ref):
    deg = degp_ref[0, :, 0:1] + degp_ref[1, :, 0:1] + 1.0
    dinv = lax.rsqrt(deg)
    g0_ref[...] = dinv * x_ref[...]
    dinv_ref[...] = dinv


def _combine_body(p_ref, g_ref, dinv_ref, out_ref):
    dinv = dinv_ref[...]
    out_ref[...] = dinv * dinv * (p_ref[0] + p_ref[1] + g_ref[...])


def _final_body(p_ref, g_ref, dinv_ref, w_ref, b_ref, out_ref):
    h2 = dinv_ref[...] * (p_ref[0] + p_ref[1] + g_ref[...])
    out_ref[...] = lax.dot_general(
        h2, w_ref[...], (((1,), (1,)), ((), ())),
        preferred_element_type=jnp.float32) + b_ref[...]


def kernel(x, edge_index, W, b):
    src = edge_index[0]
    dst = edge_index[1]
    pad = E_PAD - E
    srcp = jnp.concatenate([src, jnp.zeros((pad,), jnp.int32)])
    dstp = jnp.concatenate([dst, jnp.full((pad,), N, jnp.int32)])
    ones16 = jnp.ones((CH, 16), jnp.float32)
    zeros16 = jnp.zeros((RPT, 16), jnp.float32)
    zerosD = jnp.zeros((RPT, D), jnp.float32)

    degp = _deg_kernel(dstp, ones16, zeros16)

    row_spec = pl.BlockSpec((BLK, D), lambda i: (i, 0))
    part_spec = pl.BlockSpec((NC, BLK, D), lambda i: (0, i, 0))
    dinv_spec = pl.BlockSpec((BLK, 1), lambda i: (i, 0))

    g0, dinv = pl.pallas_call(
        _scale_init_body,
        grid=(GRID,),
        in_specs=[row_spec, pl.BlockSpec((NC, BLK, 16), lambda i: (0, i, 0))],
        out_specs=[row_spec, dinv_spec],
        out_shape=[jax.ShapeDtypeStruct((N, D), jnp.float32),
                   jax.ShapeDtypeStruct((N, 1), jnp.float32)],
    )(x, degp)

    p1 = _hop_kernel(g0, srcp, dstp, zerosD)

    g1 = pl.pallas_call(
        _combine_body,
        grid=(GRID,),
        in_specs=[part_spec, row_spec, dinv_spec],
        out_specs=row_spec,
        out_shape=jax.ShapeDtypeStruct((N, D), jnp.float32),
    )(p1, g0, dinv)

    p2 = _hop_kernel(g1, srcp, dstp, zerosD)

    out = pl.pallas_call(
        _final_body,
        grid=(GRID,),
        in_specs=[part_spec, row_spec, dinv_spec,
                  pl.BlockSpec((D, D), lambda i: (0, 0)),
                  pl.BlockSpec((1, D), lambda i: (0, 0))],
        out_specs=row_spec,
        out_shape=jax.ShapeDtypeStruct((N, D), jnp.float32),
    )(p2, g1, dinv, W, b.reshape(1, D))
    return out


# speedup vs baseline: 10.3003x; 10.3003x over previous
import functools

import jax
import jax.numpy as jnp
from jax import lax
from jax.experimental import pallas as pl
from jax.experimental.pallas import tpu as pltpu
from jax.experimental.pallas import tpu_sc as plsc

N = 10000
E = 320000
D = 128

NC = 2
NS = 16
NW = NC * NS

CH = 128
EPW = -(-E // (NW * CH)) * CH
E_PAD = EPW * NW
NCHUNK = EPW // CH

NPAD = N + 112
RPT = NPAD // NS

BLK = 400
GRID = N // BLK

_mesh = plsc.VectorSubcoreMesh(core_axis_name="c", subcore_axis_name="s")


@functools.partial(
    pl.kernel,
    out_type=jax.ShapeDtypeStruct((NC, NPAD, 16), jnp.float32),
    mesh=_mesh,
    scratch_types=[
        pltpu.VMEM((1, CH), jnp.int32),
        pltpu.VMEM((CH, 16), jnp.float32),
        pltpu.VMEM_SHARED((NPAD, 16), jnp.float32),
        pltpu.SemaphoreType.DMA,
    ],
)
def _deg_kernel(dst_hbm, ones_hbm, zeros_hbm, out_hbm, didx, ones_v, acc, sem):
    c = lax.axis_index("c")
    s = lax.axis_index("s")
    wid = c * NS + s
    pltpu.sync_copy(ones_hbm, ones_v)
    pltpu.sync_copy(zeros_hbm, acc.at[pl.ds(s * RPT, RPT), :])
    plsc.subcore_barrier()

    def body(ch, carry):
        off = wid * EPW + ch * CH
        pltpu.sync_copy(dst_hbm.at[pl.ds(off, CH)], didx.at[0])
        pltpu.sync_copy(ones_v, acc.at[didx.at[0]], add=True)
        return carry

    lax.fori_loop(0, NCHUNK, body, 0)
    plsc.subcore_barrier()
    pltpu.sync_copy(acc.at[pl.ds(s * RPT, RPT), :],
                    out_hbm.at[c, pl.ds(s * RPT, RPT), :])


@functools.partial(
    pl.kernel,
    out_type=jax.ShapeDtypeStruct((NC, NPAD, D), jnp.float32),
    mesh=_mesh,
    scratch_types=[
        pltpu.VMEM((1, CH), jnp.int32),
        pltpu.VMEM((1, CH), jnp.int32),
        pltpu.VMEM((CH, D), jnp.float32),
        pltpu.VMEM_SHARED((NPAD, D), jnp.float32),
        pltpu.SemaphoreType.DMA,
    ],
)
def _hop_kernel(g_hbm, src_hbm, dst_hbm, zeros_hbm, out_hbm,
                sidx, didx, rows, acc, sem):
    c = lax.axis_index("c")
    s = lax.axis_index("s")
    wid = c * NS + s
    pltpu.sync_copy(zeros_hbm, acc.at[pl.ds(s * RPT, RPT), :])
    plsc.subcore_barrier()

    def body(ch, carry):
        off = wid * EPW + ch * CH
        pltpu.sync_copy(src_hbm.at[pl.ds(off, CH)], sidx.at[0])
        pltpu.sync_copy(dst_hbm.at[pl.ds(off, CH)], didx.at[0])
        pltpu.async_copy(g_hbm.at[sidx.at[0]], rows, sem).wait()
        pltpu.sync_copy(rows, acc.at[didx.at[0]], add=True)
        return carry

    lax.fori_loop(0, NCHUNK, body, 0)
    plsc.subcore_barrier()
    pltpu.sync_copy(acc.at[pl.ds(s * RPT, RPT), :],
                    out_hbm.at[c, pl.ds(s * RPT, RPT), :])


def _scale_init_body(x_ref, degp_ref, g0_ref, dinv_ref):
    deg = degp_ref[0, :, 0:1] + degp_ref[1, :, 0:1] + 1.0
    dinv = lax.rsqrt(deg)
    g0_ref[...] = dinv * x_ref[...]
    dinv_ref[...] = dinv


def _combine_body(p_ref, g_ref, dinv_ref, out_ref):
    dinv = dinv_ref[...]
    out_ref[...] = dinv * dinv * (p_ref[0] + p_ref[1] + g_ref[...])


def _final_body(p_ref, g_ref, dinv_ref, w_ref, b_ref, out_ref):
    h2 = dinv_ref[...] * (p_ref[0] + p_ref[1] + g_ref[...])
    out_ref[...] = lax.dot_general(
        h2, w_ref[...], (((1,), (1,)), ((), ())),
        preferred_element_type=jnp.float32) + b_ref[...]


def kernel(x, edge_index, W, b):
    src = edge_index[0]
    dst = edge_index[1]
    pad = E_PAD - E
    srcp = jnp.concatenate([src, jnp.zeros((pad,), jnp.int32)])
    dstp = jnp.concatenate([dst, jnp.full((pad,), N, jnp.int32)])
    ones16 = jnp.ones((CH, 16), jnp.float32)
    zeros16 = jnp.zeros((RPT, 16), jnp.float32)
    zerosD = jnp.zeros((RPT, D), jnp.float32)

    degp = _deg_kernel(dstp, ones16, zeros16)

    row_spec = pl.BlockSpec((BLK, D), lambda i: (i, 0))
    part_spec = pl.BlockSpec((NC, BLK, D), lambda i: (0, i, 0))
    dinv_spec = pl.BlockSpec((BLK, 1), lambda i: (i, 0))

    g0, dinv = pl.pallas_call(
        _scale_init_body,
        grid=(GRID,),
        in_specs=[row_spec, pl.BlockSpec((NC, BLK, 16), lambda i: (0, i, 0))],
        out_specs=[row_spec, dinv_spec],
        out_shape=[jax.ShapeDtypeStruct((N, D), jnp.float32),
                   jax.ShapeDtypeStruct((N, 1), jnp.float32)],
    )(x, degp)

    p1 = _hop_kernel(g0, srcp, dstp, zerosD)

    g1 = pl.pallas_call(
        _combine_body,
        grid=(GRID,),
        in_specs=[part_spec, row_spec, dinv_spec],
        out_specs=row_spec,
        out_shape=jax.ShapeDtypeStruct((N, D), jnp.float32),
    )(p1, g0, dinv)

    p2 = _hop_kernel(g1, srcp, dstp, zerosD)

    out = pl.pallas_call(
        _final_body,
        grid=(GRID,),
        in_specs=[part_spec, row_spec, dinv_spec,
                  pl.BlockSpec((D, D), lambda i: (0, 0)),
                  pl.BlockSpec((1, D), lambda i: (0, 0))],
        out_specs=row_spec,
        out_shape=jax.ShapeDtypeStruct((N, D), jnp.float32),
    )(p2, g1, dinv, W, b.reshape(1, D))
    return out
